# baseline (speedup 1.0000x reference)
import functools

import jax
import jax.numpy as jnp
from jax.experimental import pallas as pl
from jax.experimental.pallas import tpu as pltpu

ALPHA = 0.25
GAMMA = 2.0
DIVIDE_LINE = 1.0 / 9.0


def _loss_kernel(anchors_ref, cls_ref, reg_ref, boxes_ref, labels_ref,
                 csum_ref, rsum_ref, npos_ref, *, blk):
    j = pl.program_id(1)

    @pl.when(j == 0)
    def _init():
        csum_ref[...] = jnp.zeros_like(csum_ref)
        rsum_ref[...] = jnp.zeros_like(rsum_ref)
        npos_ref[...] = jnp.zeros_like(npos_ref)

    anc = anchors_ref[...]
    boxes = boxes_ref[0]
    labels = labels_ref[0]

    ax0 = anc[:, 0:1]
    ay0 = anc[:, 1:2]
    ax1 = anc[:, 2:3]
    ay1 = anc[:, 3:4]
    bx0 = boxes[:, 0].reshape(1, -1)
    by0 = boxes[:, 1].reshape(1, -1)
    bx1 = boxes[:, 2].reshape(1, -1)
    by1 = boxes[:, 3].reshape(1, -1)

    area_a = (ax1 - ax0) * (ay1 - ay0)
    area_b = (bx1 - bx0) * (by1 - by0)
    ltx = jnp.maximum(ax0, bx0)
    lty = jnp.maximum(ay0, by0)
    rbx = jnp.minimum(ax1, bx1)
    rby = jnp.minimum(ay1, by1)
    wx = jnp.clip(rbx - ltx, 0.0)
    wy = jnp.clip(rby - lty, 0.0)
    inter = wx * wy
    union = area_a + area_b - inter
    iou = inter / jnp.maximum(union, 1e-09)

    iou_max = jnp.max(iou, axis=1, keepdims=True)
    m_iota = jax.lax.broadcasted_iota(jnp.int32, iou.shape, 1)
    big = jnp.int32(1 << 30)
    matched = jnp.min(jnp.where(iou == iou_max, m_iota, big),
                      axis=1, keepdims=True)
    sel = m_iota == matched

    matched_label = jnp.sum(jnp.where(sel, labels, 0.0), axis=1,
                            keepdims=True)
    pos = iou_max >= 0.5
    ign = jnp.logical_and(iou_max >= 0.4, iou_max < 0.5)
    posf = pos.astype(jnp.float32)

    cls = cls_ref[0]
    p = jnp.clip(cls, 1e-06, 1.0 - 1e-06)
    bg = (1.0 - ALPHA) * -jnp.log(1.0 - p) * p * p
    c_iota = jax.lax.broadcasted_iota(jnp.int32, p.shape, 1)
    at_m = c_iota == matched_label.astype(jnp.int32)
    p_m = jnp.sum(jnp.where(at_m, p, 0.0), axis=1, keepdims=True)
    q_m = 1.0 - p_m
    bg_m = (1.0 - ALPHA) * -jnp.log(q_m) * p_m * p_m
    fg_m = ALPHA * -jnp.log(p_m) * q_m * q_m
    corr = posf * (fg_m - bg_m) - ign.astype(jnp.float32) * bg_m
    csum_ref[0] += jnp.full((1, 128), jnp.sum(bg) + jnp.sum(corr))
    npos_ref[0] += jnp.full((1, 128), jnp.sum(posf))

    sel_f = sel.astype(jnp.float32)
    gx0 = jnp.sum(sel_f * bx0, axis=1, keepdims=True)
    gy0 = jnp.sum(sel_f * by0, axis=1, keepdims=True)
    gx1 = jnp.sum(sel_f * bx1, axis=1, keepdims=True)
    gy1 = jnp.sum(sel_f * by1, axis=1, keepdims=True)

    aw = ax1 - ax0
    ah = ay1 - ay0
    acx = ax0 + 0.5 * aw
    acy = ay0 + 0.5 * ah
    gw = gx1 - gx0
    gh = gy1 - gy0
    gcx = gx0 + 0.5 * gw
    gcy = gy0 + 0.5 * gh
    dx = (gcx - acx) / aw
    dy = (gcy - acy) / ah
    dw = jnp.log(gw / aw)
    dh = jnp.log(gh / ah)
    reg_true = jnp.concatenate([dx, dy, dw, dh], axis=1)

    reg = reg_ref[0]
    diff = jnp.abs(reg - reg_true)
    sl = jnp.where(diff < DIVIDE_LINE,
                   0.5 / DIVIDE_LINE * diff * diff,
                   diff - 0.5 * DIVIDE_LINE)
    rsum_ref[0] += jnp.full((1, 128), jnp.sum(sl * posf))


@jax.jit
def kernel(classifications, regressions, anchors, gt_boxes, gt_labels):
    n, a, c = classifications.shape
    m = gt_boxes.shape[1]
    blk = 2000
    grid = (n, a // blk)

    labels_f = gt_labels.astype(jnp.float32).reshape(n, 1, m)

    out_shape = [jax.ShapeDtypeStruct((n, 1, 128), jnp.float32)] * 3
    csum, rsum, npos = pl.pallas_call(
        functools.partial(_loss_kernel, blk=blk),
        grid=grid,
        in_specs=[
            pl.BlockSpec((blk, 4), lambda i, j: (j, 0)),
            pl.BlockSpec((1, blk, c), lambda i, j: (i, j, 0)),
            pl.BlockSpec((1, blk, 4), lambda i, j: (i, j, 0)),
            pl.BlockSpec((1, m, 4), lambda i, j: (i, 0, 0)),
            pl.BlockSpec((1, 1, m), lambda i, j: (i, 0, 0)),
        ],
        out_specs=[
            pl.BlockSpec((1, 1, 128), lambda i, j: (i, 0, 0)),
            pl.BlockSpec((1, 1, 128), lambda i, j: (i, 0, 0)),
            pl.BlockSpec((1, 1, 128), lambda i, j: (i, 0, 0)),
        ],
        out_shape=out_shape,
    )(anchors, classifications, regressions, gt_boxes, labels_f)

    csum = csum[:, 0, 0]
    rsum = rsum[:, 0, 0]
    npos = npos[:, 0, 0]
    denom = jnp.maximum(npos, 1.0)
    class_loss = jnp.mean(csum / denom)
    reg_loss = jnp.mean(jnp.where(npos > 0, rsum / (denom * 4.0), 0.0))
    return (class_loss, reg_loss)


# speedup vs baseline: 2.6855x; 2.6855x over previous
import functools

import jax
import jax.numpy as jnp
from jax import lax
from jax.experimental import pallas as pl
from jax.experimental.pallas import tpu as pltpu
from jax.experimental.pallas import tpu_sc as plsc

ALPHA = 0.25
GAMMA = 2.0
DIVIDE_LINE = 1.0 / 9.0

LANE = 128
ROWS = 784
RB = 112
NW = 32
GCHUNK = 49


def _match_kernel(boxes_ref, labels_ref, anc_ref, reg_ref,
                  posf_ref, ignf_ref, idx_ref, rsum_ref, npos_ref,
                  *, n_anchors, n_classes, n_boxes):
    i = pl.program_id(0)
    j = pl.program_id(1)

    @pl.when(j == 0)
    def _init():
        rsum_ref[...] = jnp.zeros_like(rsum_ref)
        npos_ref[...] = jnp.zeros_like(npos_ref)

    ax0 = anc_ref[0]
    ay0 = anc_ref[1]
    ax1 = anc_ref[2]
    ay1 = anc_ref[3]
    area_a = (ax1 - ax0) * (ay1 - ay0)

    best_iou = jnp.full(ax0.shape, -1.0, jnp.float32)
    best_lab = jnp.zeros(ax0.shape, jnp.float32)
    bgx0 = jnp.zeros(ax0.shape, jnp.float32)
    bgy0 = jnp.zeros(ax0.shape, jnp.float32)
    bgx1 = jnp.zeros(ax0.shape, jnp.float32)
    bgy1 = jnp.zeros(ax0.shape, jnp.float32)

    for m in range(n_boxes):
        bx0 = boxes_ref[0, 0, 4 * m + 0]
        by0 = boxes_ref[0, 0, 4 * m + 1]
        bx1 = boxes_ref[0, 0, 4 * m + 2]
        by1 = boxes_ref[0, 0, 4 * m + 3]
        area_b = (bx1 - bx0) * (by1 - by0)
        wx = jnp.clip(jnp.minimum(ax1, bx1) - jnp.maximum(ax0, bx0), 0.0)
        wy = jnp.clip(jnp.minimum(ay1, by1) - jnp.maximum(ay0, by0), 0.0)
        inter = wx * wy
        union = area_a + area_b - inter
        iou = inter / jnp.maximum(union, 1e-09)
        upd = iou > best_iou
        best_iou = jnp.where(upd, iou, best_iou)
        lab = labels_ref[0, 0, m]
        best_lab = jnp.where(upd, lab, best_lab)
        bgx0 = jnp.where(upd, bx0, bgx0)
        bgy0 = jnp.where(upd, by0, bgy0)
        bgx1 = jnp.where(upd, bx1, bgx1)
        bgy1 = jnp.where(upd, by1, bgy1)

    a_idx = (jax.lax.broadcasted_iota(jnp.int32, ax0.shape, 0) * LANE
             + jax.lax.broadcasted_iota(jnp.int32, ax0.shape, 1)
             + j * (RB * LANE))
    valid = a_idx < n_anchors
    validf = valid.astype(jnp.float32)

    posf = (best_iou >= 0.5).astype(jnp.float32) * validf
    ignf = (jnp.logical_and(best_iou >= 0.4, best_iou < 0.5)
            .astype(jnp.float32) * validf)
    posf_ref[0] = posf
    ignf_ref[0] = ignf
    flat = (i * (n_anchors * n_classes) + a_idx * n_classes
            + best_lab.astype(jnp.int32))
    idx_ref[0] = jnp.where(valid, flat, 0)
    npos_ref[0] += jnp.sum(posf, axis=0, keepdims=True)

    aw = ax1 - ax0
    ah = ay1 - ay0
    acx = ax0 + 0.5 * aw
    acy = ay0 + 0.5 * ah
    gw = bgx1 - bgx0
    gh = bgy1 - bgy0
    gcx = bgx0 + 0.5 * gw
    gcy = bgy0 + 0.5 * gh
    t0 = (gcx - acx) / aw
    t1 = (gcy - acy) / ah
    t2 = jnp.log(gw / aw)
    t3 = jnp.log(gh / ah)

    sl_acc = jnp.zeros(ax0.shape, jnp.float32)
    for k, t in enumerate((t0, t1, t2, t3)):
        diff = jnp.abs(reg_ref[0, k] - t)
        sl_acc += jnp.where(diff < DIVIDE_LINE,
                            0.5 / DIVIDE_LINE * diff * diff,
                            diff - 0.5 * DIVIDE_LINE)
    rsum_ref[0] += jnp.sum(sl_acc * posf, axis=0, keepdims=True)


def _dense_kernel(cls_ref, bsum_ref):
    j = pl.program_id(1)

    @pl.when(j == 0)
    def _init():
        bsum_ref[...] = jnp.zeros_like(bsum_ref)

    p = jnp.clip(cls_ref[0], 1e-06, 1.0 - 1e-06)
    bg = (1.0 - ALPHA) * -jnp.log(1.0 - p) * p * p
    bsum_ref[0] += jnp.sum(bg, axis=0, keepdims=True)


def _corr_kernel(pam_ref, posf_ref, ignf_ref, csum_ref):
    p = jnp.clip(pam_ref[0], 1e-06, 1.0 - 1e-06)
    q = 1.0 - p
    bg_m = (1.0 - ALPHA) * -jnp.log(q) * p * p
    fg_m = ALPHA * -jnp.log(p) * q * q
    corr = posf_ref[0] * (fg_m - bg_m) - ignf_ref[0] * bg_m
    csum_ref[0] = jnp.sum(corr, axis=0, keepdims=True)


def _gather_kernel(table_ref, idx_hbm_ref, out_ref, idx_v, rows_v, sem):
    wid = lax.axis_index("s") * 2 + lax.axis_index("c")
    pltpu.sync_copy(idx_hbm_ref.at[wid], idx_v)
    pltpu.async_copy(table_ref.at[idx_v], rows_v, sem).wait()
    pltpu.sync_copy(rows_v, out_ref.at[wid])


def _sc_gather(cls_flat, idx):
    mesh = plsc.VectorSubcoreMesh(core_axis_name="c", subcore_axis_name="s")
    fn = functools.partial(
        pl.kernel,
        out_type=jax.ShapeDtypeStruct((NW, GCHUNK * LANE), jnp.float32),
        mesh=mesh,
        scratch_types=[
            pltpu.VMEM((GCHUNK * LANE,), jnp.int32),
            pltpu.VMEM((GCHUNK * LANE,), jnp.float32),
            pltpu.SemaphoreType.DMA,
        ],
    )(_gather_kernel)
    return fn(cls_flat, idx)


@jax.jit
def kernel(classifications, regressions, anchors, gt_boxes, gt_labels):
    n, a, c = classifications.shape
    m = gt_boxes.shape[1]
    a_pad = ROWS * LANE

    anc_pad = jnp.concatenate(
        [anchors,
         jnp.broadcast_to(jnp.array([0.0, 0.0, 1.0, 1.0], jnp.float32),
                          (a_pad - a, 4))], axis=0)
    anc_t = anc_pad.T.reshape(4, ROWS, LANE)
    reg_pad = jnp.pad(regressions, ((0, 0), (0, a_pad - a), (0, 0)))
    reg_t = reg_pad.transpose(0, 2, 1).reshape(n, 4, ROWS, LANE)
    boxes_s = gt_boxes.reshape(n, 1, 4 * m)
    labels_s = gt_labels.astype(jnp.float32).reshape(n, 1, m)

    posf, ignf, idxm, rsum, npos = pl.pallas_call(
        functools.partial(_match_kernel, n_anchors=a, n_classes=c,
                          n_boxes=m),
        grid=(n, ROWS // RB),
        in_specs=[
            pl.BlockSpec((1, 1, 4 * m), lambda i, j: (i, 0, 0),
                         memory_space=pltpu.MemorySpace.SMEM),
            pl.BlockSpec((1, 1, m), lambda i, j: (i, 0, 0),
                         memory_space=pltpu.MemorySpace.SMEM),
            pl.BlockSpec((4, RB, LANE), lambda i, j: (0, j, 0)),
            pl.BlockSpec((1, 4, RB, LANE), lambda i, j: (i, 0, j, 0)),
        ],
        out_specs=[
            pl.BlockSpec((1, RB, LANE), lambda i, j: (i, j, 0)),
            pl.BlockSpec((1, RB, LANE), lambda i, j: (i, j, 0)),
            pl.BlockSpec((1, RB, LANE), lambda i, j: (i, j, 0)),
            pl.BlockSpec((1, 1, LANE), lambda i, j: (i, 0, 0)),
            pl.BlockSpec((1, 1, LANE), lambda i, j: (i, 0, 0)),
        ],
        out_shape=[
            jax.ShapeDtypeStruct((n, ROWS, LANE), jnp.float32),
            jax.ShapeDtypeStruct((n, ROWS, LANE), jnp.float32),
            jax.ShapeDtypeStruct((n, ROWS, LANE), jnp.int32),
            jax.ShapeDtypeStruct((n, 1, LANE), jnp.float32),
            jax.ShapeDtypeStruct((n, 1, LANE), jnp.float32),
        ],
    )(boxes_s, labels_s, anc_t, reg_t)

    pam = _sc_gather(classifications.reshape(-1),
                     idxm.reshape(NW, GCHUNK * LANE)).reshape(n, ROWS, LANE)

    corr = pl.pallas_call(
        _corr_kernel,
        grid=(n,),
        in_specs=[
            pl.BlockSpec((1, ROWS, LANE), lambda i: (i, 0, 0)),
            pl.BlockSpec((1, ROWS, LANE), lambda i: (i, 0, 0)),
            pl.BlockSpec((1, ROWS, LANE), lambda i: (i, 0, 0)),
        ],
        out_specs=pl.BlockSpec((1, 1, LANE), lambda i: (i, 0, 0)),
        out_shape=jax.ShapeDtypeStruct((n, 1, LANE), jnp.float32),
    )(pam, posf, ignf)

    db = 5000
    bsum = pl.pallas_call(
        _dense_kernel,
        grid=(n, a // db),
        in_specs=[pl.BlockSpec((1, db, c), lambda i, j: (i, j, 0))],
        out_specs=pl.BlockSpec((1, 1, c), lambda i, j: (i, 0, 0)),
        out_shape=jax.ShapeDtypeStruct((n, 1, c), jnp.float32),
    )(classifications)

    csum = jnp.sum(bsum, axis=(1, 2)) + jnp.sum(corr, axis=(1, 2))
    np_ = jnp.sum(npos, axis=(1, 2))
    rs = jnp.sum(rsum, axis=(1, 2))
    denom = jnp.maximum(np_, 1.0)
    class_loss = jnp.mean(csum / denom)
    reg_loss = jnp.mean(jnp.where(np_ > 0, rs / (denom * 4.0), 0.0))
    return (class_loss, reg_loss)
